# Initial kernel scaffold; baseline (speedup 1.0000x reference)
import jax
import jax.numpy as jnp
from jax.experimental import pallas as pl


def kernel(visited_node_score, selected_edges, visited_node_representation, rel_emb, query_src_ts_emb, query_rel_emb, Wq, Wk, W_lin, b_lin, max_edges):
    raise NotImplementedError("write your pallas kernel here")


# speedup vs baseline: 1.0951x; 1.0951x over previous
import jax
import jax.numpy as jnp
from jax.experimental import pallas as pl

_D = 128


def _logits_body(hsrc_ref, hdst_ref, rel_ref, w1q_ref, w2q_ref, w1k_ref,
                 w2k_ref, aq_ref, bq_ref, out_ref):
    hsrc = hsrc_ref[0]
    hdst = hdst_ref[0]
    rel = rel_ref[0]
    a = (jnp.dot(hsrc, w1q_ref[...], preferred_element_type=jnp.float32)
         + jnp.dot(rel, w2q_ref[...], preferred_element_type=jnp.float32)
         + aq_ref[0])
    b = (jnp.dot(hdst, w1k_ref[...], preferred_element_type=jnp.float32)
         + jnp.dot(rel, w2k_ref[...], preferred_element_type=jnp.float32)
         + bq_ref[0])
    out_ref[...] = jnp.sum(a * b, axis=1)[None, None, :]


def _final_body(agg_ref, h_ref, deg_ref, wl_ref, bl_ref, out_ref):
    x = jnp.where(deg_ref[...] > 0.0, agg_ref[...], h_ref[...])
    y = jnp.dot(x, wl_ref[...].T, preferred_element_type=jnp.float32)
    y = y + bl_ref[...]
    out_ref[...] = jnp.where(y >= 0.0, y, 0.01 * y)


def kernel(visited_node_score, selected_edges, visited_node_representation,
           rel_emb, query_src_ts_emb, query_rel_emb, Wq, Wk, W_lin, b_lin,
           max_edges):
    H = visited_node_representation
    N = H.shape[0]
    E = selected_edges.shape[0]
    Q = query_src_ts_emb.shape[0]
    PER_Q = E // Q
    D = _D

    qidx = selected_edges[:, 0]
    src = selected_edges[:, 6]
    dst = selected_edges[:, 7]

    Cq = jnp.concatenate([query_src_ts_emb, query_rel_emb], axis=1)
    W1qT = Wq[:, :D].T
    W2qT = Wq[:, D:2 * D].T
    W1kT = Wk[:, :D].T
    W2kT = Wk[:, D:2 * D].T
    Aq = Cq @ Wq[:, 2 * D:].T
    Bq = Cq @ Wk[:, 2 * D:].T

    hsrc = H[src].reshape(Q, PER_Q, D)
    hdst = H[dst].reshape(Q, PER_Q, D)
    rel3 = rel_emb.reshape(Q, PER_Q, D)

    qblk3 = lambda i: (i, 0, 0)
    full = lambda i: (0, 0)
    logits = pl.pallas_call(
        _logits_body,
        grid=(Q,),
        in_specs=[
            pl.BlockSpec((1, PER_Q, D), qblk3),
            pl.BlockSpec((1, PER_Q, D), qblk3),
            pl.BlockSpec((1, PER_Q, D), qblk3),
            pl.BlockSpec((D, 4 * D), full),
            pl.BlockSpec((D, 4 * D), full),
            pl.BlockSpec((D, 4 * D), full),
            pl.BlockSpec((D, 4 * D), full),
            pl.BlockSpec((1, 1, 4 * D), qblk3),
            pl.BlockSpec((1, 1, 4 * D), qblk3),
        ],
        out_specs=pl.BlockSpec((1, 1, PER_Q), qblk3),
        out_shape=jax.ShapeDtypeStruct((Q, 1, PER_Q), jnp.float32),
    )(hsrc, hdst, rel3, W1qT, W2qT, W1kT, W2kT,
      Aq.reshape(Q, 1, 4 * D), Bq.reshape(Q, 1, 4 * D))
    logits = logits.reshape(E)

    mx = jax.ops.segment_max(logits, src, num_segments=N)
    mx = jnp.where(jnp.isfinite(mx), mx, 0.0)
    ex = jnp.exp(logits - mx[src])
    s = jax.ops.segment_sum(ex, src, num_segments=N)
    soft = ex / (s[src] + 1e-16)

    src_score = visited_node_score[src]
    target_score = soft * src_score

    ts = target_score.reshape(Q, PER_Q)
    _, topi = jax.lax.top_k(ts, 500)
    orig_indices = (jnp.arange(Q, dtype=jnp.int32)[:, None] * PER_Q
                    + topi).reshape(-1)

    p_src = src[orig_indices]
    p_dst = dst[orig_indices]
    p_soft = soft[orig_indices]
    p_src_score = src_score[orig_indices]

    updated_node_score = jax.ops.segment_sum(p_soft * p_src_score, p_dst,
                                             num_segments=N)
    agg = jax.ops.segment_sum(p_soft[:, None] * H[p_dst], p_src,
                              num_segments=N)
    deg = jax.ops.segment_sum(jnp.ones_like(p_soft), p_src, num_segments=N)

    BN = 2000
    node = lambda i: (i, 0)
    updated_repr = pl.pallas_call(
        _final_body,
        grid=(N // BN,),
        in_specs=[
            pl.BlockSpec((BN, D), node),
            pl.BlockSpec((BN, D), node),
            pl.BlockSpec((BN, 1), node),
            pl.BlockSpec((D, D), full),
            pl.BlockSpec((1, D), full),
        ],
        out_specs=pl.BlockSpec((BN, D), node),
        out_shape=jax.ShapeDtypeStruct((N, D), jnp.float32),
    )(agg, H, deg[:, None], W_lin, b_lin[None, :])

    return updated_node_score, updated_repr, orig_indices


# speedup vs baseline: 1.2556x; 1.1466x over previous
import functools

import jax
import jax.numpy as jnp
from jax import lax
from jax.experimental import pallas as pl
from jax.experimental.pallas import tpu as pltpu
from jax.experimental.pallas import tpu_sc as plsc

_D = 128


def _sc_gather_rows(table, src, dst):
    E = src.shape[0]
    D = table.shape[1]
    NW = 32
    per_w = E // NW
    CH = 1000
    nch = per_w // CH
    mesh = plsc.VectorSubcoreMesh(core_axis_name="c", subcore_axis_name="s")

    @functools.partial(
        pl.kernel, mesh=mesh,
        out_type=[jax.ShapeDtypeStruct((E, D), jnp.float32),
                  jax.ShapeDtypeStruct((E, D), jnp.float32)],
        scratch_types=[pltpu.VMEM((CH,), jnp.int32),
                       pltpu.VMEM((CH, D), jnp.float32),
                       pltpu.SemaphoreType.DMA],
    )
    def k(table_hbm, src_hbm, dst_hbm, osrc_hbm, odst_hbm, idx_v, rows_v,
          sem):
        wid = lax.axis_index("s") * 2 + lax.axis_index("c")
        base = wid * per_w
        for ihbm, ohbm in ((src_hbm, osrc_hbm), (dst_hbm, odst_hbm)):
            for c in range(nch):
                off = base + c * CH
                pltpu.sync_copy(ihbm.at[pl.ds(off, CH)], idx_v)
                pltpu.async_copy(table_hbm.at[idx_v], rows_v, sem).wait()
                pltpu.sync_copy(rows_v, ohbm.at[pl.ds(off, CH)])

    return k(table, src, dst)


def _logits_body(hsrc_ref, hdst_ref, rel_ref, w1q_ref, w2q_ref, w1k_ref,
                 w2k_ref, aq_ref, bq_ref, out_ref):
    hsrc = hsrc_ref[0]
    hdst = hdst_ref[0]
    rel = rel_ref[0]
    a = (jnp.dot(hsrc, w1q_ref[...], preferred_element_type=jnp.float32)
         + jnp.dot(rel, w2q_ref[...], preferred_element_type=jnp.float32)
         + aq_ref[0])
    b = (jnp.dot(hdst, w1k_ref[...], preferred_element_type=jnp.float32)
         + jnp.dot(rel, w2k_ref[...], preferred_element_type=jnp.float32)
         + bq_ref[0])
    out_ref[...] = jnp.sum(a * b, axis=1)[None, None, :]


def _final_body(agg_ref, h_ref, deg_ref, wl_ref, bl_ref, out_ref):
    x = jnp.where(deg_ref[...] > 0.0, agg_ref[...], h_ref[...])
    y = jnp.dot(x, wl_ref[...].T, preferred_element_type=jnp.float32)
    y = y + bl_ref[...]
    out_ref[...] = jnp.where(y >= 0.0, y, 0.01 * y)


def kernel(visited_node_score, selected_edges, visited_node_representation,
           rel_emb, query_src_ts_emb, query_rel_emb, Wq, Wk, W_lin, b_lin,
           max_edges):
    H = visited_node_representation
    N = H.shape[0]
    E = selected_edges.shape[0]
    Q = query_src_ts_emb.shape[0]
    PER_Q = E // Q
    D = _D

    qidx = selected_edges[:, 0]
    src = selected_edges[:, 6]
    dst = selected_edges[:, 7]

    Cq = jnp.concatenate([query_src_ts_emb, query_rel_emb], axis=1)
    W1qT = Wq[:, :D].T
    W2qT = Wq[:, D:2 * D].T
    W1kT = Wk[:, :D].T
    W2kT = Wk[:, D:2 * D].T
    Aq = Cq @ Wq[:, 2 * D:].T
    Bq = Cq @ Wk[:, 2 * D:].T

    hsrc2, hdst2 = _sc_gather_rows(H, src, dst)
    hsrc = hsrc2.reshape(Q, PER_Q, D)
    hdst = hdst2.reshape(Q, PER_Q, D)
    rel3 = rel_emb.reshape(Q, PER_Q, D)

    qblk3 = lambda i: (i, 0, 0)
    full = lambda i: (0, 0)
    logits = pl.pallas_call(
        _logits_body,
        grid=(Q,),
        in_specs=[
            pl.BlockSpec((1, PER_Q, D), qblk3),
            pl.BlockSpec((1, PER_Q, D), qblk3),
            pl.BlockSpec((1, PER_Q, D), qblk3),
            pl.BlockSpec((D, 4 * D), full),
            pl.BlockSpec((D, 4 * D), full),
            pl.BlockSpec((D, 4 * D), full),
            pl.BlockSpec((D, 4 * D), full),
            pl.BlockSpec((1, 1, 4 * D), qblk3),
            pl.BlockSpec((1, 1, 4 * D), qblk3),
        ],
        out_specs=pl.BlockSpec((1, 1, PER_Q), qblk3),
        out_shape=jax.ShapeDtypeStruct((Q, 1, PER_Q), jnp.float32),
    )(hsrc, hdst, rel3, W1qT, W2qT, W1kT, W2kT,
      Aq.reshape(Q, 1, 4 * D), Bq.reshape(Q, 1, 4 * D))
    logits = logits.reshape(E)

    mx = jax.ops.segment_max(logits, src, num_segments=N)
    mx = jnp.where(jnp.isfinite(mx), mx, 0.0)
    ex = jnp.exp(logits - mx[src])
    s = jax.ops.segment_sum(ex, src, num_segments=N)
    soft = ex / (s[src] + 1e-16)

    src_score = visited_node_score[src]
    target_score = soft * src_score

    ts = target_score.reshape(Q, PER_Q)
    _, topi = jax.lax.top_k(ts, 500)
    orig_indices = (jnp.arange(Q, dtype=jnp.int32)[:, None] * PER_Q
                    + topi).reshape(-1)

    p_src = src[orig_indices]
    p_dst = dst[orig_indices]
    p_soft = soft[orig_indices]
    p_src_score = src_score[orig_indices]

    updated_node_score = jax.ops.segment_sum(p_soft * p_src_score, p_dst,
                                             num_segments=N)
    agg = jax.ops.segment_sum(p_soft[:, None] * H[p_dst], p_src,
                              num_segments=N)
    deg = jax.ops.segment_sum(jnp.ones_like(p_soft), p_src, num_segments=N)

    BN = 2000
    node = lambda i: (i, 0)
    updated_repr = pl.pallas_call(
        _final_body,
        grid=(N // BN,),
        in_specs=[
            pl.BlockSpec((BN, D), node),
            pl.BlockSpec((BN, D), node),
            pl.BlockSpec((BN, 1), node),
            pl.BlockSpec((D, D), full),
            pl.BlockSpec((1, D), full),
        ],
        out_specs=pl.BlockSpec((BN, D), node),
        out_shape=jax.ShapeDtypeStruct((N, D), jnp.float32),
    )(agg, H, deg[:, None], W_lin, b_lin[None, :])

    return updated_node_score, updated_repr, orig_indices
